# Initial kernel scaffold; baseline (speedup 1.0000x reference)
import jax
import jax.numpy as jnp
from jax.experimental import pallas as pl


def kernel(points_coords, points_features, ref_coords, ref_features, ref_t_embed):
    raise NotImplementedError("write your pallas kernel here")


# speedup vs baseline: 23.4640x; 23.4640x over previous
import functools

import jax
import jax.numpy as jnp
from jax.experimental import pallas as pl
from jax.experimental.pallas import tpu as pltpu

B, N, M = 4, 8192, 1024
C_REF, C_SKIP, C_T = 256, 128, 128
NB = 512


def _body(ct_ref, rc_ref, rf_ref, rt_ref, pf_ref, feat_ref, temb_ref):
    ct = ct_ref[0]
    rc = rc_ref[0]

    dx = ct[0:1, :] - rc[:, 0:1]
    dy = ct[1:2, :] - rc[:, 1:2]
    dz = ct[2:3, :] - rc[:, 2:3]
    d2 = dx * dx + dy * dy + dz * dz

    iota_m = jax.lax.broadcasted_iota(jnp.int32, (M, NB), 0)
    idxs = []
    dists = []
    for k in range(3):
        dk = jnp.min(d2, axis=0, keepdims=True)
        ik = jnp.min(jnp.where(d2 == dk, iota_m, M), axis=0, keepdims=True)
        idxs.append(ik)
        dists.append(dk)
        if k < 2:
            d2 = jnp.where(iota_m == ik, jnp.float32(jnp.inf), d2)

    w0 = 1.0 / jnp.maximum(dists[0], 1e-10)
    w1 = 1.0 / jnp.maximum(dists[1], 1e-10)
    w2 = 1.0 / jnp.maximum(dists[2], 1e-10)
    s = w0 + w1 + w2
    w0, w1, w2 = w0 / s, w1 / s, w2 / s

    wm = (
        jnp.where(iota_m == idxs[0], w0, 0.0)
        + jnp.where(iota_m == idxs[1], w1, 0.0)
        + jnp.where(iota_m == idxs[2], w2, 0.0)
    )

    dot = functools.partial(
        jax.lax.dot_general,
        dimension_numbers=(((1,), (0,)), ((), ())),
        precision=jax.lax.Precision.HIGHEST,
        preferred_element_type=jnp.float32,
    )
    feat_ref[0, 0:C_REF, :] = dot(rf_ref[0], wm)
    feat_ref[0, C_REF:, :] = pf_ref[0]
    temb_ref[0] = dot(rt_ref[0], wm)


def kernel(points_coords, points_features, ref_coords, ref_features, ref_t_embed):
    coords_t = jnp.transpose(points_coords, (0, 2, 1))
    grid = (B, N // NB)
    out = pl.pallas_call(
        _body,
        grid=grid,
        in_specs=[
            pl.BlockSpec((1, 3, NB), lambda b, n: (b, 0, n)),
            pl.BlockSpec((1, M, 3), lambda b, n: (b, 0, 0)),
            pl.BlockSpec((1, C_REF, M), lambda b, n: (b, 0, 0)),
            pl.BlockSpec((1, C_T, M), lambda b, n: (b, 0, 0)),
            pl.BlockSpec((1, C_SKIP, NB), lambda b, n: (b, 0, n)),
        ],
        out_specs=[
            pl.BlockSpec((1, C_REF + C_SKIP, NB), lambda b, n: (b, 0, n)),
            pl.BlockSpec((1, C_T, NB), lambda b, n: (b, 0, n)),
        ],
        out_shape=[
            jax.ShapeDtypeStruct((B, C_REF + C_SKIP, N), jnp.float32),
            jax.ShapeDtypeStruct((B, C_T, N), jnp.float32),
        ],
    )(coords_t, ref_coords, ref_features, ref_t_embed, points_features)
    return out[0], out[1]
